# baseline (speedup 1.0000x reference)
import functools

import jax
import jax.numpy as jnp
from jax import lax
from jax.experimental import pallas as pl
from jax.experimental.pallas import tpu as pltpu, tpu_sc as plsc

_B = 16384
_C = 1024
_BIT = 64
_BLK = 1024
_NB = _B // _BLK
_Q = _BLK // 4
_NCHUNK = _BIT // 16
_PAD = 128

_INFO = plsc.get_sparse_core_info()
_NC, _NS = _INFO.num_cores, _INFO.num_subcores
_NW = _NC * _NS
_SPW = _B // _NW
_GW = 128
_ROWS_PT = _C // _NS


@functools.lru_cache(maxsize=None)
def _rnd_pm1():
    r = jax.random.randint(jax.random.key(1), (_B, _BIT), 0, 2)
    return r.astype(jnp.float32) * 2.0 - 1.0


def _argmax_body(y1, y2, y3, y4, out_ref):
    iota_c = jax.lax.broadcasted_iota(jnp.int32, (_Q, _C), 1)
    parts = []
    for y_ref in (y1, y2, y3, y4):
        vals = y_ref[...]
        m = jnp.max(vals, axis=1, keepdims=True)
        parts.append(jnp.min(jnp.where(vals == m, iota_c, _C), axis=1))
    out_ref[...] = jnp.concatenate(parts)[None, None, :]


def _tc_labels(y):
    out = pl.pallas_call(
        _argmax_body,
        grid=(_NB,),
        in_specs=[
            pl.BlockSpec((_Q, _C), lambda i: (4 * i, 0)),
            pl.BlockSpec((_Q, _C), lambda i: (4 * i + 1, 0)),
            pl.BlockSpec((_Q, _C), lambda i: (4 * i + 2, 0)),
            pl.BlockSpec((_Q, _C), lambda i: (4 * i + 3, 0)),
        ],
        out_specs=pl.BlockSpec((1, 1, _BLK), lambda i: (i, 0, 0)),
        out_shape=jax.ShapeDtypeStruct((_NB, 1, _BLK), jnp.int32),
    )(y, y, y, y)
    return out.reshape(_B)


def _sc_scatter(labels, h, rnd, codebook):
    mesh = plsc.VectorSubcoreMesh(core_axis_name="c", subcore_axis_name="s")

    @functools.partial(
        pl.kernel,
        mesh=mesh,
        out_type=(jax.ShapeDtypeStruct((_C, _PAD), jnp.float32),
                  jax.ShapeDtypeStruct((_C, _PAD), jnp.float32),
                  jax.ShapeDtypeStruct((_C, _PAD), jnp.float32),
                  jax.ShapeDtypeStruct((_C, _PAD), jnp.float32)),
        scratch_types=[
            pltpu.VMEM((_GW,), jnp.int32),
            pltpu.VMEM((_GW, _BIT), jnp.float32),
            pltpu.VMEM((_GW, _BIT), jnp.float32),
            pltpu.VMEM((_GW, _PAD), jnp.float32),
            pltpu.VMEM((_GW, _PAD), jnp.float32),
            pltpu.VMEM((_ROWS_PT, _PAD), jnp.float32),
            pltpu.VMEM((_ROWS_PT, _BIT), jnp.float32),
            pltpu.VMEM_SHARED((_C, _PAD), jnp.float32),
            pltpu.VMEM_SHARED((_C, _PAD), jnp.float32),
        ],
    )
    def k(lab_hbm, h_hbm, rnd_hbm, cb_hbm, d0_hbm, d1_hbm, e0_hbm, e1_hbm,
          lab_v, h_v, rnd_v, u1_v, u2_v, row_v, cb_v, t1, t2):
        cid = lax.axis_index("c")
        sid = lax.axis_index("s")
        wid = sid * _NC + cid
        base = wid * _SPW

        zero = jnp.zeros((16,), jnp.float32)
        one = jnp.full((16,), 1.0, jnp.float32)
        seed = cid == 0

        pltpu.sync_copy(cb_hbm.at[pl.ds(sid * _ROWS_PT, _ROWS_PT), :], cb_v)

        def zrow(r, carry):
            for c in range(_NCHUNK):
                sl = pl.ds(c * 16, 16)
                row_v[r, sl] = jnp.where(seed, cb_v[r, sl], zero)
            for c in range(_NCHUNK, _PAD // 16):
                row_v[r, pl.ds(c * 16, 16)] = zero
            return carry

        lax.fori_loop(0, _ROWS_PT, zrow, 0)
        rows = pl.ds(sid * _ROWS_PT, _ROWS_PT)
        pltpu.sync_copy(row_v, t1.at[rows, :])

        def zrow2(r, carry):
            for c in range(_PAD // 16):
                row_v[r, pl.ds(c * 16, 16)] = zero
            return carry

        lax.fori_loop(0, _ROWS_PT, zrow2, 0)
        pltpu.sync_copy(row_v, t2.at[rows, :])
        plsc.subcore_barrier()

        for w in range(_SPW // _GW):
            wbase = base + w * _GW
            pltpu.sync_copy(lab_hbm.at[pl.ds(wbase, _GW)], lab_v)
            pltpu.sync_copy(h_hbm.at[pl.ds(wbase, _GW), :], h_v)
            pltpu.sync_copy(rnd_hbm.at[pl.ds(wbase, _GW), :], rnd_v)

            def srow(r, carry):
                for c in range(_NCHUNK):
                    sl = pl.ds(c * 16, 16)
                    sl2 = pl.ds(_BIT + c * 16, 16)
                    v = h_v[r, sl]
                    u1_v[r, sl] = jnp.where(v > 0.0, one,
                                            jnp.where(v < 0.0, -one, zero))
                    u1_v[r, sl2] = v
                    u2_v[r, sl] = v * rnd_v[r, sl]
                    u2_v[r, sl2] = v * v
                return carry

            lax.fori_loop(0, _GW, srow, 0)
            pltpu.sync_copy(u1_v, t1.at[lab_v], add=True)
            pltpu.sync_copy(u2_v, t2.at[lab_v], add=True)

        plsc.subcore_barrier()

        @pl.when(cid == 0)
        def _():
            pltpu.sync_copy(t1.at[rows, :], d0_hbm.at[rows, :])
            pltpu.sync_copy(t2.at[rows, :], e0_hbm.at[rows, :])

        @pl.when(cid == 1)
        def _():
            pltpu.sync_copy(t1.at[rows, :], d1_hbm.at[rows, :])
            pltpu.sync_copy(t2.at[rows, :], e1_hbm.at[rows, :])

    return k(labels, h, rnd, codebook)


def _epi_body(d0, d1, e0, e1, out_ref):
    a1 = d0[...] + d1[...]
    a2 = e0[...] + e1[...]
    t = a1[:, :_BIT]
    s_sum = a1[:, _BIT:]
    r_sum = a2[:, :_BIT]
    h2 = jnp.sum(a2[:, _BIT:])
    dot = (jnp.sum(s_sum * jnp.sign(t))
           + jnp.sum(jnp.where(t == 0.0, r_sum, 0.0)))
    loss = h2 * 0.5 + (_B * _BIT) * 0.5 - dot
    out_ref[...] = jnp.full((1, 1), loss, jnp.float32)


def _tc_epilogue(d0, d1, e0, e1):
    return pl.pallas_call(
        _epi_body,
        out_shape=jax.ShapeDtypeStruct((1, 1), jnp.float32),
    )(d0, d1, e0, e1)


def kernel(h, y, codebook, alpha):
    rnd = _rnd_pm1()
    labels = _tc_labels(y)
    d0, d1, e0, e1 = _sc_scatter(labels, h, rnd, codebook)
    out = _tc_epilogue(d0, d1, e0, e1)
    return out[0, 0] * alpha


# speedup vs baseline: 1.0459x; 1.0459x over previous
import functools

import jax
import jax.numpy as jnp
from jax import lax
from jax.experimental import pallas as pl
from jax.experimental.pallas import tpu as pltpu, tpu_sc as plsc

_B = 16384
_C = 1024
_BIT = 64
_BLK = 1024
_NB = _B // _BLK
_Q = _BLK // 4
_NCHUNK = _BIT // 16
_PAD = 128

_INFO = plsc.get_sparse_core_info()
_NC, _NS = _INFO.num_cores, _INFO.num_subcores
_NW = _NC * _NS
_SPW = _B // _NW
_GW = 256
_ROWS_PT = _C // _NS


@functools.lru_cache(maxsize=None)
def _rnd_pm1():
    r = jax.random.randint(jax.random.key(1), (_B, _BIT), 0, 2)
    return r.astype(jnp.float32) * 2.0 - 1.0


def _argmax_body(y1, y2, y3, y4, out_ref):
    iota_c = jax.lax.broadcasted_iota(jnp.int32, (_Q, _C), 1)
    parts = []
    for y_ref in (y1, y2, y3, y4):
        vals = y_ref[...]
        m = jnp.max(vals, axis=1, keepdims=True)
        parts.append(jnp.min(jnp.where(vals == m, iota_c, _C), axis=1))
    out_ref[...] = jnp.concatenate(parts)[None, None, :]


def _tc_labels(y):
    out = pl.pallas_call(
        _argmax_body,
        grid=(_NB,),
        in_specs=[
            pl.BlockSpec((_Q, _C), lambda i: (4 * i, 0)),
            pl.BlockSpec((_Q, _C), lambda i: (4 * i + 1, 0)),
            pl.BlockSpec((_Q, _C), lambda i: (4 * i + 2, 0)),
            pl.BlockSpec((_Q, _C), lambda i: (4 * i + 3, 0)),
        ],
        out_specs=pl.BlockSpec((1, 1, _BLK), lambda i: (i, 0, 0)),
        out_shape=jax.ShapeDtypeStruct((_NB, 1, _BLK), jnp.int32),
    )(y, y, y, y)
    return out.reshape(_B)


def _sc_scatter(labels, h, codebook):
    mesh = plsc.VectorSubcoreMesh(core_axis_name="c", subcore_axis_name="s")

    @functools.partial(
        pl.kernel,
        mesh=mesh,
        out_type=(jax.ShapeDtypeStruct((_C, _PAD), jnp.float32),
                  jax.ShapeDtypeStruct((_C, _PAD), jnp.float32)),
        scratch_types=[
            pltpu.VMEM((_GW,), jnp.int32),
            pltpu.VMEM((_GW, _BIT), jnp.float32),
            pltpu.VMEM((_GW, _PAD), jnp.float32),
            pltpu.VMEM((_ROWS_PT, _PAD), jnp.float32),
            pltpu.VMEM((_ROWS_PT, _BIT), jnp.float32),
            pltpu.VMEM_SHARED((_C, _PAD), jnp.float32),
        ],
    )
    def k(lab_hbm, h_hbm, cb_hbm, d0_hbm, d1_hbm,
          lab_v, h_v, sgn_v, row_v, cb_v, table):
        cid = lax.axis_index("c")
        sid = lax.axis_index("s")
        wid = sid * _NC + cid
        base = wid * _SPW

        zero = jnp.zeros((16,), jnp.float32)
        one = jnp.full((16,), 1.0, jnp.float32)
        seed = cid == 0

        pltpu.sync_copy(cb_hbm.at[pl.ds(sid * _ROWS_PT, _ROWS_PT), :], cb_v)

        def zrow(r, carry):
            for c in range(_NCHUNK):
                sl = pl.ds(c * 16, 16)
                row_v[r, sl] = jnp.where(seed, cb_v[r, sl], zero)
            for c in range(_NCHUNK, _PAD // 16):
                row_v[r, pl.ds(c * 16, 16)] = zero
            return carry

        lax.fori_loop(0, _ROWS_PT, zrow, 0)
        pltpu.sync_copy(row_v, table.at[pl.ds(sid * _ROWS_PT, _ROWS_PT), :])

        def zpad(r, carry):
            for c in range(_NCHUNK, _PAD // 16):
                sgn_v[r, pl.ds(c * 16, 16)] = zero
            return carry

        lax.fori_loop(0, _GW, zpad, 0)
        plsc.subcore_barrier()

        for w in range(_SPW // _GW):
            wbase = base + w * _GW
            pltpu.sync_copy(lab_hbm.at[pl.ds(wbase, _GW)], lab_v)
            pltpu.sync_copy(h_hbm.at[pl.ds(wbase, _GW), :], h_v)

            def srow(r, carry):
                for c in range(_NCHUNK):
                    v = h_v[r, pl.ds(c * 16, 16)]
                    s = jnp.where(v > 0.0, one,
                                  jnp.where(v < 0.0, -one, zero))
                    sgn_v[r, pl.ds(c * 16, 16)] = s
                return carry

            lax.fori_loop(0, _GW, srow, 0)
            pltpu.sync_copy(sgn_v, table.at[lab_v], add=True)

        plsc.subcore_barrier()

        rows = pl.ds(sid * _ROWS_PT, _ROWS_PT)

        @pl.when(cid == 0)
        def _():
            pltpu.sync_copy(table.at[rows, :], d0_hbm.at[rows, :])

        @pl.when(cid == 1)
        def _():
            pltpu.sync_copy(table.at[rows, :], d1_hbm.at[rows, :])

    return k(labels, h, codebook)


def _sc_gather_loss(labels, h, rnd, d0, d1):
    gw = 128
    mesh = plsc.VectorSubcoreMesh(core_axis_name="c", subcore_axis_name="s")

    @functools.partial(
        pl.kernel,
        mesh=mesh,
        out_type=jax.ShapeDtypeStruct((_NW, 16), jnp.float32),
        scratch_types=[
            pltpu.VMEM((gw,), jnp.int32),
            pltpu.VMEM((gw, _PAD), jnp.float32),
            pltpu.VMEM((gw, _PAD), jnp.float32),
            pltpu.VMEM((gw, _BIT), jnp.float32),
            pltpu.VMEM((gw, _BIT), jnp.float32),
            pltpu.VMEM((16,), jnp.float32),
            pltpu.SemaphoreType.DMA,
            pltpu.SemaphoreType.DMA,
            pltpu.SemaphoreType.DMA,
            pltpu.SemaphoreType.DMA,
        ],
    )
    def k(lab_hbm, h_hbm, rnd_hbm, d0_hbm, d1_hbm, out_hbm,
          lab_v, t0_v, t1_v, h_v, rnd_v, acc_v, sem, sem2, sem3, sem4):
        cid = lax.axis_index("c")
        sid = lax.axis_index("s")
        wid = sid * _NC + cid
        base = wid * _SPW

        one = jnp.full((16,), 1.0, jnp.float32)
        acc_v[pl.ds(0, 16)] = jnp.zeros((16,), jnp.float32)

        for w in range(_SPW // gw):
            wbase = base + w * gw
            pltpu.sync_copy(lab_hbm.at[pl.ds(wbase, gw)], lab_v)
            cp_h = pltpu.async_copy(h_hbm.at[pl.ds(wbase, gw), :], h_v, sem2)
            cp_r = pltpu.async_copy(rnd_hbm.at[pl.ds(wbase, gw), :], rnd_v, sem3)
            cp0 = pltpu.async_copy(d0_hbm.at[lab_v], t0_v, sem)
            cp1 = pltpu.async_copy(d1_hbm.at[lab_v], t1_v, sem4)
            cp_h.wait()
            cp_r.wait()
            cp0.wait()
            cp1.wait()

            def lrow(r, acc):
                for c in range(_NCHUNK):
                    sl = pl.ds(c * 16, 16)
                    t = t0_v[r, sl] + t1_v[r, sl]
                    s = jnp.where(t > 0.0, one,
                                  jnp.where(t < 0.0, -one, rnd_v[r, sl]))
                    d = h_v[r, sl] - s
                    acc = acc + d * d
                return acc

            acc = lax.fori_loop(0, gw, lrow, acc_v[pl.ds(0, 16)])
            acc_v[pl.ds(0, 16)] = acc

        pltpu.sync_copy(acc_v, out_hbm.at[wid])

    return k(labels, h, rnd, d0, d1)


def kernel(h, y, codebook, alpha):
    rnd = _rnd_pm1()
    labels = _tc_labels(y)
    d0, d1 = _sc_scatter(labels, h, codebook)
    partials = _sc_gather_loss(labels, h, rnd, d0, d1)
    return jnp.sum(partials) * 0.5 * alpha
